# baseline (speedup 1.0000x reference)
import jax
import jax.numpy as jnp
from jax import lax
from jax.experimental import pallas as pl
from jax.experimental.pallas import tpu as pltpu, tpu_sc as plsc

_B = 16384
_D = 64
_NC = 2
_NS = 16
_NW = _NC * _NS
_BPW = _B // _NW
_L = 16
_NCHUNK = _BPW // _L
_NSLOT = 6
_LOOKAHEAD = 5
_TMPD = 64


def _strip_off(v):
    return pl.multiple_of(lax.shift_left(lax.shift_right_logical(v, 7), 7), 128)


def _slot_rows(slot):
    return pl.ds(pl.multiple_of(slot * _D, _D), _D)


def _gather_body(su_hbm, pu_hbm, fu_hbm, slu_hbm,
                 si_hbm, pi_hbm, fi_hbm, sli_hbm,
                 guT, giT, gu1_out, gi1_out,
                 su_v, pu_v, fu_v, slu_v,
                 si_v, pi_v, fi_v, sli_v,
                 strip_u, strip_i, tmp_u, tmp_i,
                 sem_u, sem_i, sem_out):
    c = lax.axis_index("c")
    s = lax.axis_index("s")
    wid = s * _NC + c
    base = wid * _BPW

    for hbm, v in ((su_hbm, su_v), (pu_hbm, pu_v), (fu_hbm, fu_v),
                   (slu_hbm, slu_v), (si_hbm, si_v), (pi_hbm, pi_v),
                   (fi_hbm, fi_v), (sli_hbm, sli_v)):
        pltpu.sync_copy(hbm.at[wid], v.at[pl.ds(0, _BPW)])

    feat = lax.iota(jnp.int32, _L)

    def issue(table, strips, sems, idxval, flagval, slotval):
        @pl.when(flagval == 1)
        def _():
            pltpu.async_copy(
                table.at[:, pl.ds(_strip_off(idxval), 128)],
                strips.at[_slot_rows(slotval)],
                sems.at[slotval])

    def wait(table, strips, sems, flagval, slotval):
        @pl.when(flagval == 1)
        def _():
            pltpu.make_async_copy(
                table.at[:, pl.ds(0, 128)],
                strips.at[_slot_rows(slotval)],
                sems.at[slotval]).wait()

    uv0 = su_v[pl.ds(0, _L)]
    fu0 = fu_v[pl.ds(0, _L)]
    sl0 = slu_v[pl.ds(0, _L)]
    iv0 = si_v[pl.ds(0, _L)]
    fi0 = fi_v[pl.ds(0, _L)]
    sli0 = sli_v[pl.ds(0, _L)]
    for j in range(_LOOKAHEAD):
        issue(guT, strip_u, sem_u, uv0[j], fu0[j], sl0[j])
        issue(giT, strip_i, sem_i, iv0[j], fi0[j], sli0[j])

    def chunk(k, carry):
        sl = pl.ds(k * _L, _L)
        nsl = pl.ds((k + 1) * _L, _L)
        uvec, puv, fuv, sluv = su_v[sl], pu_v[sl], fu_v[sl], slu_v[sl]
        ivec, piv, fiv, sliv = si_v[sl], pi_v[sl], fi_v[sl], sli_v[sl]
        nuvec, nfuv, nsluv = su_v[nsl], fu_v[nsl], slu_v[nsl]
        nivec, nfiv, nsliv = si_v[nsl], fi_v[nsl], sli_v[nsl]
        tj = (lax.rem(k, _TMPD // _L)) * _L
        for j in range(_L):
            r = k * _L + j
            wait(guT, strip_u, sem_u, fuv[j], sluv[j])
            wait(giT, strip_i, sem_i, fiv[j], sliv[j])
            cu = jnp.full((_L,), lax.bitwise_and(uvec[j], 127), jnp.int32)
            ci = jnp.full((_L,), lax.bitwise_and(ivec[j], 127), jnp.int32)
            fu_base = sluv[j] * _D
            fi_base = sliv[j] * _D
            for q in range(_D // _L):
                fq = feat + (q * _L)
                tmp_u[tj + j, pl.ds(q * _L, _L)] = plsc.load_gather(
                    strip_u, [fu_base + fq, cu])
                tmp_i[tj + j, pl.ds(q * _L, _L)] = plsc.load_gather(
                    strip_i, [fi_base + fq, ci])
            pltpu.async_copy(
                tmp_u.at[tj + j],
                gu1_out.at[pl.ds(pl.multiple_of(
                    lax.shift_left(puv[j], 6), _D), _D)],
                sem_out)
            pltpu.async_copy(
                tmp_i.at[tj + j],
                gi1_out.at[pl.ds(pl.multiple_of(
                    lax.shift_left(piv[j], 6), _D), _D)],
                sem_out)
            jn = j + _LOOKAHEAD
            if jn < _L:
                issue(guT, strip_u, sem_u, uvec[jn], fuv[jn], sluv[jn])
                issue(giT, strip_i, sem_i, ivec[jn], fiv[jn], sliv[jn])
            else:
                jw = jn - _L

                @pl.when(k < _NCHUNK - 1)
                def _():
                    issue(guT, strip_u, sem_u, nuvec[jw], nfuv[jw], nsluv[jw])
                    issue(giT, strip_i, sem_i, nivec[jw], nfiv[jw], nsliv[jw])
        return carry

    lax.fori_loop(0, _NCHUNK, chunk, 0)

    def drain(k, carry):
        pltpu.make_async_copy(
            tmp_u.at[0], gu1_out.at[pl.ds(0, _D)], sem_out).wait()
        return carry

    lax.fori_loop(0, 2 * _BPW, drain, 0)


def _dot_body(gu1, gi1, xui_out, ub, ib, xui_v):
    c = lax.axis_index("c")
    s = lax.axis_index("s")
    wid = s * _NC + c
    base = wid * _BPW

    pltpu.sync_copy(gu1.at[pl.ds(base * _D, _BPW * _D)], ub)
    pltpu.sync_copy(gi1.at[pl.ds(base * _D, _BPW * _D)], ib)

    lanes = lax.iota(jnp.int32, _L)

    def chunk(k, carry):
        acc = jnp.full((_L,), 0.0, jnp.float32)
        for j in range(_L):
            ro = pl.multiple_of((k * _L + j) * _D, _D)
            p = ub[pl.ds(ro, _L)] * ib[pl.ds(ro, _L)]
            for q in range(1, _D // _L):
                p = p + (ub[pl.ds(ro + q * _L, _L)]
                         * ib[pl.ds(ro + q * _L, _L)])
            acc = jnp.where(lanes == j, jnp.sum(p), acc)
        xui_v[pl.ds(k * _L, _L)] = acc
        return carry

    lax.fori_loop(0, _NCHUNK, chunk, 0)
    pltpu.sync_copy(xui_v, xui_out.at[pl.ds(base, _BPW)])


@jax.jit
def _sage_call(su, pu, fu, slu, si, pi, fi, sli, GuT, GiT):
    mesh = plsc.VectorSubcoreMesh(core_axis_name="c", subcore_axis_name="s")
    params = pltpu.CompilerParams(
        needs_layout_passes=False, disable_bounds_checks=True)
    idx_scratch = [pltpu.VMEM((_BPW + _L,), jnp.int32) for _ in range(8)]
    ga = pl.kernel(
        _gather_body,
        mesh=mesh,
        compiler_params=params,
        out_type=(
            jax.ShapeDtypeStruct((_B * _D,), jnp.float32),
            jax.ShapeDtypeStruct((_B * _D,), jnp.float32),
        ),
        scratch_types=idx_scratch + [
            pltpu.VMEM((_NSLOT * _D, 128), jnp.float32),
            pltpu.VMEM((_NSLOT * _D, 128), jnp.float32),
            pltpu.VMEM((_TMPD, _D), jnp.float32),
            pltpu.VMEM((_TMPD, _D), jnp.float32),
            pltpu.SemaphoreType.DMA((_NSLOT,)),
            pltpu.SemaphoreType.DMA((_NSLOT,)),
            pltpu.SemaphoreType.DMA,
        ],
    )
    gu1, gi1 = ga(su, pu, fu, slu, si, pi, fi, sli, GuT, GiT)
    dot = pl.kernel(
        _dot_body,
        mesh=mesh,
        compiler_params=params,
        out_type=jax.ShapeDtypeStruct((_B,), jnp.float32),
        scratch_types=[
            pltpu.VMEM((_BPW * _D,), jnp.float32),
            pltpu.VMEM((_BPW * _D,), jnp.float32),
            pltpu.VMEM((_BPW,), jnp.float32),
        ],
    )
    xui = dot(gu1, gi1)
    return xui, gu1, gi1


def _prep(idx):
    iot = lax.iota(jnp.int32, _B)
    sk, perm = lax.sort_key_val(idx.astype(jnp.int32), iot)
    so = lax.shift_right_logical(sk, 7)
    prev = jnp.concatenate([jnp.full((1,), -1, jnp.int32), so[:-1]])
    fresh = (so != prev) | (iot % _BPW == 0)
    flag = fresh.astype(jnp.int32)
    slot = (jnp.cumsum(flag) - 1) % _NSLOT
    rs = lambda a: a.reshape(_NW, _BPW)
    return rs(sk), rs(perm), rs(flag), rs(slot.astype(jnp.int32))


def kernel(user, item, Gu, Gi):
    su, pu, fu, slu = _prep(user)
    si, pi, fi, sli = _prep(item)
    xui, gu1, gi1 = _sage_call(su, pu, fu, slu, si, pi, fi, sli, Gu.T, Gi.T)
    return xui, gu1.reshape(_B, _D), gi1.reshape(_B, _D)
